# baseline (speedup 1.0000x reference)
import functools

import jax
import jax.numpy as jnp
from jax import lax
from jax.experimental import pallas as pl
from jax.experimental.pallas import tpu as pltpu
from jax.experimental.pallas import tpu_sc as plsc

N = 10000
E = 320000
D = 128
DE = 16
DU = 32
H = 128

_PREC = lax.Precision.DEFAULT

_NC = 2
_NS = 16
_NW = _NC * _NS

_GW = 128
_CH = 1280
_BE = 4000


def _dot(a, b):
    return jnp.dot(a, b, precision=_PREC, preferred_element_type=jnp.float32)


def _tables_body(x_ref, wrow_ref, wcol_ref, wu_ref, u_ref, be1_ref,
                 xr_ref, xc_ref):
    c_e = _dot(u_ref[...], wu_ref[...]) + be1_ref[...]
    xr_ref[...] = (_dot(x_ref[...], wrow_ref[...]) + c_e).astype(jnp.bfloat16)
    xc_ref[...] = _dot(x_ref[...], wcol_ref[...]).astype(jnp.bfloat16)


def _make_tables(x, w_row, w_col, w_u, u, b_e1):
    return pl.pallas_call(
        _tables_body,
        out_shape=(jax.ShapeDtypeStruct((N, H), jnp.bfloat16),
                   jax.ShapeDtypeStruct((N, H), jnp.bfloat16)),
    )(x, w_row, w_col, w_u, u, b_e1)


def _gather_rows(xr1, xc1, edge_index):
    mesh = plsc.VectorSubcoreMesh(core_axis_name="core",
                                  subcore_axis_name="subcore")

    @functools.partial(
        pl.kernel,
        out_type=jax.ShapeDtypeStruct((E, H), jnp.bfloat16),
        mesh=mesh,
        compiler_params=pltpu.CompilerParams(use_tc_tiling_on_sc=False),
    )
    def k(xr_hbm, xc_hbm, ei_hbm, g_hbm):
        def body(idx_vmem, g_vmem):
            pltpu.sync_copy(xr_hbm.at[idx_vmem.at[0]], g_vmem)
            pltpu.sync_copy(xc_hbm.at[idx_vmem.at[1]], g_vmem, add=True)

        pltpu.emit_pipeline(
            body,
            grid=(E // _GW,),
            in_specs=[pl.BlockSpec((2, _GW), lambda i: (0, i))],
            out_specs=[pl.BlockSpec((_GW, H), lambda i: (i, 0))],
            core_axis_name=("core", "subcore"),
            dimension_semantics=(pltpu.PARALLEL,),
        )(ei_hbm, g_hbm)

    return k(xr1, xc1, edge_index)


def _edge_body(g_ref, ea_ref, wea_ref, we2_ref, be2_ref,
               e2_ref, esum_ref):
    t = _dot(ea_ref[...], wea_ref[...])
    h = jnp.maximum(g_ref[...].astype(jnp.float32) + t, 0.0)
    e2 = _dot(h, we2_ref[...]) + be2_ref[...]
    e2_ref[...] = e2

    @pl.when(pl.program_id(0) == 0)
    def _():
        esum_ref[...] = jnp.zeros_like(esum_ref)

    esum_ref[...] += jnp.sum(e2, axis=0, keepdims=True)


def _edge_mlp(g, edge_attr, w_ea, w_e2, b_e2):
    grid = (E // _BE,)
    return pl.pallas_call(
        _edge_body,
        grid=grid,
        in_specs=[
            pl.BlockSpec((_BE, H), lambda i: (i, 0)),
            pl.BlockSpec((_BE, DE), lambda i: (i, 0)),
            pl.BlockSpec((DE, H), lambda i: (0, 0)),
            pl.BlockSpec((H, DE), lambda i: (0, 0)),
            pl.BlockSpec((1, DE), lambda i: (0, 0)),
        ],
        out_specs=[
            pl.BlockSpec((_BE, DE), lambda i: (i, 0)),
            pl.BlockSpec((1, DE), lambda i: (0, 0)),
        ],
        out_shape=(jax.ShapeDtypeStruct((E, DE), jnp.float32),
                   jax.ShapeDtypeStruct((1, DE), jnp.float32)),
        compiler_params=pltpu.CompilerParams(
            dimension_semantics=("arbitrary",)),
    )(g, edge_attr, w_ea, w_e2, b_e2)


def _scatter_add(e2, edge_index, zeros_nde):
    mesh = plsc.VectorSubcoreMesh(core_axis_name="core",
                                  subcore_axis_name="subcore")
    rp = 624
    tail = N - _NS * rp

    @functools.partial(
        pl.kernel,
        out_type=jax.ShapeDtypeStruct((_NC, N, DE), jnp.float32),
        mesh=mesh,
        scratch_types=[pltpu.VMEM_SHARED((N, DE), jnp.float32)],
        compiler_params=pltpu.CompilerParams(use_tc_tiling_on_sc=False),
    )
    def k(e2_hbm, ei_hbm, z_hbm, out_hbm, shared):
        c = lax.axis_index("core")
        s = lax.axis_index("subcore")
        pltpu.sync_copy(z_hbm.at[pl.ds(s * rp, rp)],
                        shared.at[pl.ds(s * rp, rp)])

        @pl.when(s == 0)
        def _():
            pltpu.sync_copy(z_hbm.at[pl.ds(_NS * rp, tail)],
                            shared.at[pl.ds(_NS * rp, tail)])

        plsc.subcore_barrier()

        def body(e2_vmem, idx_vmem):
            pltpu.sync_copy(e2_vmem, shared.at[idx_vmem.at[1]], add=True)

        pltpu.emit_pipeline(
            body,
            grid=(E // _CH,),
            in_specs=[pl.BlockSpec((_CH, DE), lambda i: (i, 0)),
                      pl.BlockSpec((2, _CH), lambda i: (0, i))],
            out_specs=[],
            core_axis_name=("core", "subcore"),
            dimension_semantics=(pltpu.PARALLEL,),
        )(e2_hbm, ei_hbm)

        plsc.subcore_barrier()
        pltpu.sync_copy(shared.at[pl.ds(s * rp, rp)],
                        out_hbm.at[c, pl.ds(s * rp, rp)])

        @pl.when(s == 0)
        def _():
            pltpu.sync_copy(shared.at[pl.ds(_NS * rp, tail)],
                            out_hbm.at[c, pl.ds(_NS * rp, tail)])

    return k(e2, edge_index, zeros_nde)


def _node_body(x_ref, aggp_ref, u_ref, esum_ref,
               wn1x_ref, wn1a_ref, wn1u_ref, bn1_ref, wn2_ref, bn2_ref,
               wg1x_ref, wg1e_ref, wg1u_ref, bg1_ref, wg2_ref, bg2_ref,
               x2_ref, u2_ref):
    agg = aggp_ref[0] + aggp_ref[1]
    c_n = _dot(u_ref[...], wn1u_ref[...]) + bn1_ref[...]
    h_n = jnp.maximum(
        _dot(x_ref[...], wn1x_ref[...]) + _dot(agg, wn1a_ref[...]) + c_n, 0.0)
    x2 = _dot(h_n, wn2_ref[...]) + bn2_ref[...]
    x2_ref[...] = x2

    xm = jnp.sum(x2, axis=0, keepdims=True) * (1.0 / N)
    em = esum_ref[...] * (1.0 / E)
    h_g = jnp.maximum(
        _dot(xm, wg1x_ref[...]) + _dot(em, wg1e_ref[...])
        + _dot(u_ref[...], wg1u_ref[...]) + bg1_ref[...], 0.0)
    u2_ref[...] = _dot(h_g, wg2_ref[...]) + bg2_ref[...]


def _node_global(x, aggp, u, esum, wn1x, wn1a, wn1u, bn1, wn2, bn2,
                 wg1x, wg1e, wg1u, bg1, wg2, bg2):
    return pl.pallas_call(
        _node_body,
        out_shape=(jax.ShapeDtypeStruct((N, D), jnp.float32),
                   jax.ShapeDtypeStruct((1, DU), jnp.float32)),
    )(x, aggp, u, esum, wn1x, wn1a, wn1u, bn1, wn2, bn2,
      wg1x, wg1e, wg1u, bg1, wg2, bg2)


def kernel(x, edge_index, edge_attr, u, v_indices, e_indices,
           W_e1, b_e1, W_e2, b_e2, W_n1, b_n1, W_n2, b_n2,
           W_g1, b_g1, W_g2, b_g2):
    w_row = W_e1[:D]
    w_col = W_e1[D:2 * D]
    w_ea = W_e1[2 * D:2 * D + DE]
    w_u = W_e1[2 * D + DE:]

    xr1, xc1 = _make_tables(x, w_row, w_col, w_u, u,
                            b_e1.reshape(1, H))
    g = _gather_rows(xr1, xc1, edge_index)
    edge_attr2, esum = _edge_mlp(g, edge_attr, w_ea, W_e2,
                                 b_e2.reshape(1, DE))
    aggp = _scatter_add(edge_attr2, edge_index,
                        jnp.zeros((N, DE), jnp.float32))
    x2, u2 = _node_global(
        x, aggp, u, esum,
        W_n1[:D], W_n1[D:D + DE], W_n1[D + DE:], b_n1.reshape(1, H),
        W_n2, b_n2.reshape(1, D),
        W_g1[:D], W_g1[D:D + DE], W_g1[D + DE:], b_g1.reshape(1, H),
        W_g2, b_g2.reshape(1, DU))
    return (x2, edge_attr2, u2)


# speedup vs baseline: 1.4632x; 1.4632x over previous
import functools

import jax
import jax.numpy as jnp
from jax import lax
from jax.experimental import pallas as pl
from jax.experimental.pallas import tpu as pltpu
from jax.experimental.pallas import tpu_sc as plsc

N = 10000
E = 320000
D = 128
DE = 16
DU = 32
H = 128

_PREC = lax.Precision.DEFAULT

_NC = 2
_NS = 16
_NW = _NC * _NS

_GW = 128
_CH = 640
_BE = 4000
_NQ = 4
_EQ = E // _NQ


def _dot(a, b):
    return jnp.dot(a, b, precision=_PREC, preferred_element_type=jnp.float32)


def _tables_body(x_ref, wrow_ref, wcol_ref, wu_ref, u_ref, be1_ref,
                 xr_ref, xc_ref):
    c_e = _dot(u_ref[...], wu_ref[...]) + be1_ref[...]
    xr_ref[...] = _dot(x_ref[...], wrow_ref[...]) + c_e
    xc_ref[...] = _dot(x_ref[...], wcol_ref[...])


def _make_tables(x, w_row, w_col, w_u, u, b_e1):
    return pl.pallas_call(
        _tables_body,
        out_shape=(jax.ShapeDtypeStruct((N, H), jnp.float32),
                   jax.ShapeDtypeStruct((N, H), jnp.float32)),
    )(x, w_row, w_col, w_u, u, b_e1)


def _gather_rows(xr1, xc1, edge_index, q):
    mesh = plsc.VectorSubcoreMesh(core_axis_name="core",
                                  subcore_axis_name="subcore")
    qoff = q * (_EQ // _GW)

    @functools.partial(
        pl.kernel,
        out_type=jax.ShapeDtypeStruct((_EQ, H), jnp.float32),
        mesh=mesh,
    )
    def k(xr_hbm, xc_hbm, ei_hbm, g_hbm):
        def body(idx_vmem, g_vmem):
            pltpu.sync_copy(xr_hbm.at[idx_vmem.at[0]], g_vmem)
            pltpu.sync_copy(xc_hbm.at[idx_vmem.at[1]], g_vmem, add=True)

        pltpu.emit_pipeline(
            body,
            grid=(_EQ // _GW,),
            in_specs=[pl.BlockSpec((2, _GW), lambda i: (0, qoff + i))],
            out_specs=[pl.BlockSpec((_GW, H), lambda i: (i, 0))],
            core_axis_name=("core", "subcore"),
            dimension_semantics=(pltpu.PARALLEL,),
        )(ei_hbm, g_hbm)

    return k(xr1, xc1, edge_index)


def _edge_body(g_ref, ea_ref, wea_ref, we2_ref, be2_ref,
               e2_ref, esum_ref):
    t = _dot(ea_ref[...], wea_ref[...])
    h = jnp.maximum(g_ref[...].astype(jnp.float32) + t, 0.0)
    e2 = _dot(h, we2_ref[...]) + be2_ref[...]
    e2_ref[...] = e2

    @pl.when(pl.program_id(0) == 0)
    def _():
        esum_ref[...] = jnp.zeros_like(esum_ref)

    esum_ref[...] += jnp.sum(e2, axis=0, keepdims=True)


def _edge_mlp(g, edge_attr, w_ea, w_e2, b_e2, q):
    nb = _EQ // _BE
    qb = q * nb
    return pl.pallas_call(
        _edge_body,
        grid=(nb,),
        in_specs=[
            pl.BlockSpec((_BE, H), lambda i: (i, 0)),
            pl.BlockSpec((_BE, DE), lambda i: (qb + i, 0)),
            pl.BlockSpec((DE, H), lambda i: (0, 0)),
            pl.BlockSpec((H, DE), lambda i: (0, 0)),
            pl.BlockSpec((1, DE), lambda i: (0, 0)),
        ],
        out_specs=[
            pl.BlockSpec((_BE, DE), lambda i: (i, 0)),
            pl.BlockSpec((1, DE), lambda i: (0, 0)),
        ],
        out_shape=(jax.ShapeDtypeStruct((_EQ, DE), jnp.float32),
                   jax.ShapeDtypeStruct((1, DE), jnp.float32)),
        compiler_params=pltpu.CompilerParams(
            dimension_semantics=("arbitrary",)),
    )(g, edge_attr, w_ea, w_e2, b_e2)


def _scatter_add(e2s, edge_index, zeros_nde):
    mesh = plsc.VectorSubcoreMesh(core_axis_name="core",
                                  subcore_axis_name="subcore")
    rp = 624
    tail = N - _NS * rp

    @functools.partial(
        pl.kernel,
        out_type=jax.ShapeDtypeStruct((_NC, N, DE), jnp.float32),
        mesh=mesh,
        scratch_types=[pltpu.VMEM_SHARED((N, DE), jnp.float32)],
        compiler_params=pltpu.CompilerParams(use_tc_tiling_on_sc=False),
    )
    def k(e2a_hbm, e2b_hbm, e2c_hbm, e2d_hbm, ei_hbm, z_hbm, out_hbm, shared):
        c = lax.axis_index("core")
        s = lax.axis_index("subcore")
        pltpu.sync_copy(z_hbm.at[pl.ds(s * rp, rp)],
                        shared.at[pl.ds(s * rp, rp)])

        @pl.when(s == 0)
        def _():
            pltpu.sync_copy(z_hbm.at[pl.ds(_NS * rp, tail)],
                            shared.at[pl.ds(_NS * rp, tail)])

        plsc.subcore_barrier()

        def body(e2_vmem, idx_vmem):
            pltpu.sync_copy(e2_vmem, shared.at[idx_vmem.at[1]], add=True)

        for q, e2_hbm in enumerate((e2a_hbm, e2b_hbm, e2c_hbm, e2d_hbm)):
            qoff = q * (_EQ // _CH)
            pltpu.emit_pipeline(
                body,
                grid=(_EQ // _CH,),
                in_specs=[pl.BlockSpec((_CH, DE), lambda i: (i, 0)),
                          pl.BlockSpec((2, _CH),
                                       lambda i, qoff=qoff: (0, qoff + i))],
                out_specs=[],
                core_axis_name=("core", "subcore"),
                dimension_semantics=(pltpu.PARALLEL,),
            )(e2_hbm, ei_hbm)

        plsc.subcore_barrier()
        pltpu.sync_copy(shared.at[pl.ds(s * rp, rp)],
                        out_hbm.at[c, pl.ds(s * rp, rp)])

        @pl.when(s == 0)
        def _():
            pltpu.sync_copy(shared.at[pl.ds(_NS * rp, tail)],
                            out_hbm.at[c, pl.ds(_NS * rp, tail)])

    return k(*e2s, edge_index, zeros_nde)


def _node_body(x_ref, aggp_ref, u_ref, esum_ref,
               wn1x_ref, wn1a_ref, wn1u_ref, bn1_ref, wn2_ref, bn2_ref,
               wg1x_ref, wg1e_ref, wg1u_ref, bg1_ref, wg2_ref, bg2_ref,
               x2_ref, u2_ref):
    agg = aggp_ref[0] + aggp_ref[1]
    c_n = _dot(u_ref[...], wn1u_ref[...]) + bn1_ref[...]
    h_n = jnp.maximum(
        _dot(x_ref[...], wn1x_ref[...]) + _dot(agg, wn1a_ref[...]) + c_n, 0.0)
    x2 = _dot(h_n, wn2_ref[...]) + bn2_ref[...]
    x2_ref[...] = x2

    xm = jnp.sum(x2, axis=0, keepdims=True) * (1.0 / N)
    em = esum_ref[...] * (1.0 / E)
    h_g = jnp.maximum(
        _dot(xm, wg1x_ref[...]) + _dot(em, wg1e_ref[...])
        + _dot(u_ref[...], wg1u_ref[...]) + bg1_ref[...], 0.0)
    u2_ref[...] = _dot(h_g, wg2_ref[...]) + bg2_ref[...]


def _node_global(x, aggp, u, esum, wn1x, wn1a, wn1u, bn1, wn2, bn2,
                 wg1x, wg1e, wg1u, bg1, wg2, bg2):
    return pl.pallas_call(
        _node_body,
        out_shape=(jax.ShapeDtypeStruct((N, D), jnp.float32),
                   jax.ShapeDtypeStruct((1, DU), jnp.float32)),
    )(x, aggp, u, esum, wn1x, wn1a, wn1u, bn1, wn2, bn2,
      wg1x, wg1e, wg1u, bg1, wg2, bg2)


def kernel(x, edge_index, edge_attr, u, v_indices, e_indices,
           W_e1, b_e1, W_e2, b_e2, W_n1, b_n1, W_n2, b_n2,
           W_g1, b_g1, W_g2, b_g2):
    w_row = W_e1[:D]
    w_col = W_e1[D:2 * D]
    w_ea = W_e1[2 * D:2 * D + DE]
    w_u = W_e1[2 * D + DE:]

    xr1, xc1 = _make_tables(x, w_row, w_col, w_u, u,
                            b_e1.reshape(1, H))
    e2s, esums = [], []
    for q in range(_NQ):
        gq = _gather_rows(xr1, xc1, edge_index, q)
        e2q, esq = _edge_mlp(gq, edge_attr, w_ea, W_e2,
                             b_e2.reshape(1, DE), q)
        e2s.append(e2q)
        esums.append(esq)
    edge_attr2 = jnp.concatenate(e2s, axis=0)
    esum = esums[0] + esums[1] + esums[2] + esums[3]
    aggp = _scatter_add(e2s, edge_index,
                        jnp.zeros((N, DE), jnp.float32))
    x2, u2 = _node_global(
        x, aggp, u, esum,
        W_n1[:D], W_n1[D:D + DE], W_n1[D + DE:], b_n1.reshape(1, H),
        W_n2, b_n2.reshape(1, D),
        W_g1[:D], W_g1[D:D + DE], W_g1[D + DE:], b_g1.reshape(1, H),
        W_g2, b_g2.reshape(1, DU))
    return (x2, edge_attr2, u2)
